# baseline (speedup 1.0000x reference)
import functools

import jax
import jax.numpy as jnp
from jax import lax
from jax.experimental import pallas as pl
from jax.experimental.pallas import tpu as pltpu

E = 8
TOP_K = 2
H = 1024
F = 1408
T = 2048
BF = 128

_NEG = -1e30


def _router_body(hs_ref, gw_ref, logits_ref, comb_ref):
    hs = hs_ref[...]
    gw = gw_ref[...]
    logits = jax.lax.dot_general(
        hs, gw, (((1,), (1,)), ((), ())),
        preferred_element_type=jnp.float32,
    )
    logits_ref[...] = logits

    lanes = lax.broadcasted_iota(jnp.int32, (T, E), 1)
    m1 = jnp.max(logits, axis=1, keepdims=True)
    i1 = jnp.min(jnp.where(logits == m1, lanes, E), axis=1, keepdims=True)
    masked = jnp.where(lanes == i1, _NEG, logits)
    m2 = jnp.max(masked, axis=1, keepdims=True)
    i2 = jnp.min(jnp.where(masked == m2, lanes, E), axis=1, keepdims=True)
    s1 = 1.0 / (1.0 + jnp.exp(m2 - m1))
    s2 = 1.0 - s1
    comb_ref[...] = jnp.where(lanes == i1, s1, 0.0) + jnp.where(lanes == i2, s2, 0.0)


def _router(hs, gate_w):
    return pl.pallas_call(
        _router_body,
        out_shape=(
            jax.ShapeDtypeStruct((T, E), jnp.float32),
            jax.ShapeDtypeStruct((T, E), jnp.float32),
        ),
    )(hs, gate_w)


def _moe_body(hs_ref, comb_ref, wg_ref, wu_ref, wd_ref, out_ref):
    e = pl.program_id(0)
    f = pl.program_id(1)

    @pl.when(jnp.logical_and(e == 0, f == 0))
    def _():
        out_ref[...] = jnp.zeros_like(out_ref)

    xb = hs_ref[...].astype(jnp.bfloat16)
    g = jax.lax.dot_general(
        xb, wg_ref[0], (((1,), (1,)), ((), ())),
        preferred_element_type=jnp.float32)
    u = jax.lax.dot_general(
        xb, wu_ref[0], (((1,), (1,)), ((), ())),
        preferred_element_type=jnp.float32)
    act = g * (1.0 / (1.0 + jnp.exp(-g))) * u
    contrib = jax.lax.dot_general(
        act.astype(jnp.bfloat16), wd_ref[0], (((1,), (1,)), ((), ())),
        preferred_element_type=jnp.float32)

    lanes = lax.broadcasted_iota(jnp.int32, (T, E), 1)
    wcol = jnp.sum(jnp.where(lanes == e, comb_ref[...], 0.0), axis=1,
                   keepdims=True)
    out_ref[...] += wcol * contrib


def _moe(hs, comb, gpw, upw, dpw):
    grid = (E, F // BF)
    return pl.pallas_call(
        _moe_body,
        grid=grid,
        in_specs=[
            pl.BlockSpec((T, H), lambda e, f: (0, 0)),
            pl.BlockSpec((T, E), lambda e, f: (0, 0)),
            pl.BlockSpec((1, BF, H), lambda e, f: (e, f, 0)),
            pl.BlockSpec((1, BF, H), lambda e, f: (e, f, 0)),
            pl.BlockSpec((1, H, BF), lambda e, f: (e, 0, f)),
        ],
        out_specs=pl.BlockSpec((T, H), lambda e, f: (0, 0)),
        out_shape=jax.ShapeDtypeStruct((T, H), jnp.float32),
    )(hs, comb, gpw, upw, dpw)


def kernel(x, gate_w, gate_proj_w, up_proj_w, down_proj_w):
    b, s, h = x.shape
    hs = x.reshape(-1, h)
    logits, comb = _router(hs, gate_w)
    final = _moe(hs, comb,
                 gate_proj_w.astype(jnp.bfloat16),
                 up_proj_w.astype(jnp.bfloat16),
                 down_proj_w.astype(jnp.bfloat16))
    return final.reshape(b, s, h), logits
